# baseline (speedup 1.0000x reference)
import functools

import jax
import jax.numpy as jnp
from jax import lax
from jax.experimental import pallas as pl
from jax.experimental.pallas import tpu as pltpu
from jax.experimental.pallas import tpu_sc as plsc

BATCH = 16384
FEAT = 64
WIDE = 128
LANES = 16
NUM_CORES = 2
NUM_SUBCORES = 16
NUM_WORKERS = NUM_CORES * NUM_SUBCORES
NUM_CLS = 1000000
HALF = NUM_CLS // 2

BPW = BATCH // NUM_WORKERS
PPW = BPW // 2
CHUNK = 128
NCHUNK = BPW // CHUNK
PAIRS = CHUNK // 2
VECS = FEAT // LANES

WCH = 1000
NW = HALF // WCH


def _widen_body(lo_ref, hi_ref, wide_ref):
    wide_ref[...] = jnp.concatenate([lo_ref[...], hi_ref[...]], axis=-1)


def _loss_body(feat_hbm, lab_hbm, msk_hbm, wide_hbm, out_hbm,
               idx_v, row_v, msk_v, cent_v, feat_v, acc_v, gsems, fsem, msem):
    wid = lax.axis_index("s") * NUM_CORES + lax.axis_index("c")
    pltpu.sync_copy(lab_hbm.at[wid], idx_v)
    fcopy = pltpu.async_copy(feat_hbm.at[pl.ds(wid * PPW, PPW), :],
                             feat_v, fsem)
    mcopy = pltpu.async_copy(msk_hbm.at[pl.ds(wid * PPW, PPW), :],
                             msk_v, msem)

    for j in range(NCHUNK):
        for v in range(CHUNK // LANES):
            lab = idx_v[j, pl.ds(v * LANES, LANES)]
            row = lab - jnp.where(lab >= HALF, jnp.int32(HALF), jnp.int32(0))
            row_v[j, pl.ds(v * LANES, LANES)] = row

    def fire(j):
        pltpu.async_copy(wide_hbm.at[row_v.at[j]], cent_v.at[j % 2],
                         gsems.at[j % 2])

    def drain(j):
        pltpu.make_async_copy(wide_hbm.at[pl.ds(0, CHUNK), :],
                              cent_v.at[j % 2], gsems.at[j % 2]).wait()

    fire(0)
    fire(1)
    fcopy.wait()
    mcopy.wait()

    zero = jnp.zeros((LANES,), jnp.float32)
    accs = (zero,) * VECS
    for j in range(NCHUNK):
        drain(j)
        b = j % 2

        def pair(i, a, j=j, b=b):
            p = j * PAIRS + i
            out = list(a)
            for h in range(2):
                for l in range(VECS):
                    off = h * FEAT + l * LANES
                    f = feat_v[p, pl.ds(off, LANES)]
                    m = msk_v[p, pl.ds(off, LANES)]
                    clo = cent_v[b, 2 * i + h, pl.ds(l * LANES, LANES)]
                    chi = cent_v[b, 2 * i + h, pl.ds(FEAT + l * LANES, LANES)]
                    dlo = f - clo
                    dhi = f - chi
                    d2 = dlo * dlo + m * (dhi * dhi - dlo * dlo)
                    out[l] = out[l] + d2
            return tuple(out)

        accs = lax.fori_loop(0, PAIRS, pair, accs)
        if j + 2 < NCHUNK:
            fire(j + 2)

    acc_v[...] = accs[0] + accs[1] + accs[2] + accs[3]
    pltpu.sync_copy(acc_v, out_hbm.at[wid])


@functools.partial(jax.jit, static_argnames=())
def _center_loss(features, labels, centers):
    labels = labels.astype(jnp.int32)
    featw = features.reshape(BATCH // 2, WIDE)
    mask = jnp.broadcast_to(
        (labels >= HALF).astype(jnp.float32)[:, None],
        (BATCH, FEAT)).reshape(BATCH // 2, WIDE)
    lab3 = labels.reshape(NUM_WORKERS, NCHUNK, CHUNK)

    widen = pl.pallas_call(
        _widen_body,
        grid=(NW,),
        in_specs=[
            pl.BlockSpec((WCH, FEAT), lambda i: (i, 0)),
            pl.BlockSpec((WCH, FEAT), lambda i: (i + NW, 0)),
        ],
        out_specs=pl.BlockSpec((WCH, WIDE), lambda i: (i, 0)),
        out_shape=jax.ShapeDtypeStruct((HALF, WIDE), jnp.float32),
    )
    wide = widen(centers, centers)

    kern = pl.kernel(
        _loss_body,
        out_type=jax.ShapeDtypeStruct((NUM_WORKERS, LANES), jnp.float32),
        mesh=plsc.VectorSubcoreMesh(core_axis_name="c", subcore_axis_name="s"),
        scratch_types=[
            pltpu.VMEM((NCHUNK, CHUNK), jnp.int32),
            pltpu.VMEM((NCHUNK, CHUNK), jnp.int32),
            pltpu.VMEM((PPW, WIDE), jnp.float32),
            pltpu.VMEM((2, CHUNK, WIDE), jnp.float32),
            pltpu.VMEM((PPW, WIDE), jnp.float32),
            pltpu.VMEM((LANES,), jnp.float32),
            pltpu.SemaphoreType.DMA((2,)),
            pltpu.SemaphoreType.DMA,
            pltpu.SemaphoreType.DMA,
        ],
    )
    partials = kern(featw, lab3, mask, wide)
    return jnp.sum(partials) / BATCH


def kernel(features, labels, centers):
    return _center_loss(features, labels, centers)


# speedup vs baseline: 1.3135x; 1.3135x over previous
import functools

import jax
import jax.numpy as jnp
from jax import lax
from jax.experimental import pallas as pl
from jax.experimental.pallas import tpu as pltpu
from jax.experimental.pallas import tpu_sc as plsc

BATCH = 16384
FEAT = 64
WIDE = 128
LANES = 16
NUM_CORES = 2
NUM_SUBCORES = 16
NUM_WORKERS = NUM_CORES * NUM_SUBCORES
NUM_CLS = 1000000
HALF = NUM_CLS // 2

BPW = BATCH // NUM_WORKERS
PPW = BPW // 2
CHUNK = 128
NCHUNK = BPW // CHUNK
PAIRS = CHUNK // 2
VECS = FEAT // LANES

WCH = 5000
NW = HALF // WCH


def _widen_body(lo_ref, hi_ref, wide_ref):
    wide_ref[...] = jnp.concatenate([lo_ref[...], hi_ref[...]], axis=-1)


def _loss_body(feat_hbm, lab_hbm, msk_hbm, wide_hbm, out_hbm,
               idx_v, row_v, msk_v, cent_v, feat_v, acc_v, gsems, fsem, msem):
    wid = lax.axis_index("s") * NUM_CORES + lax.axis_index("c")
    pltpu.sync_copy(lab_hbm.at[wid], idx_v)
    fcopy = pltpu.async_copy(feat_hbm.at[pl.ds(wid * PPW, PPW), :],
                             feat_v, fsem)
    mcopy = pltpu.async_copy(msk_hbm.at[pl.ds(wid * PPW, PPW), :],
                             msk_v, msem)

    for j in range(NCHUNK):
        for v in range(CHUNK // LANES):
            lab = idx_v[j, pl.ds(v * LANES, LANES)]
            row = lab - jnp.where(lab >= HALF, jnp.int32(HALF), jnp.int32(0))
            row_v[j, pl.ds(v * LANES, LANES)] = row

    def fire(j):
        pltpu.async_copy(wide_hbm.at[row_v.at[j]], cent_v.at[j % 2],
                         gsems.at[j % 2])

    def drain(j):
        pltpu.make_async_copy(wide_hbm.at[pl.ds(0, CHUNK), :],
                              cent_v.at[j % 2], gsems.at[j % 2]).wait()

    fire(0)
    fire(1)
    fcopy.wait()
    mcopy.wait()

    zero = jnp.zeros((LANES,), jnp.float32)
    accs = (zero,) * VECS
    for j in range(NCHUNK):
        drain(j)
        b = j % 2

        def pair(i, a, j=j, b=b):
            p = j * PAIRS + i
            out = list(a)
            for h in range(2):
                for l in range(VECS):
                    off = h * FEAT + l * LANES
                    f = feat_v[p, pl.ds(off, LANES)]
                    m = msk_v[p, pl.ds(off, LANES)]
                    clo = cent_v[b, 2 * i + h, pl.ds(l * LANES, LANES)]
                    chi = cent_v[b, 2 * i + h, pl.ds(FEAT + l * LANES, LANES)]
                    dlo = f - clo
                    dhi = f - chi
                    d2 = dlo * dlo + m * (dhi * dhi - dlo * dlo)
                    out[l] = out[l] + d2
            return tuple(out)

        accs = lax.fori_loop(0, PAIRS, pair, accs)
        if j + 2 < NCHUNK:
            fire(j + 2)

    acc_v[...] = accs[0] + accs[1] + accs[2] + accs[3]
    pltpu.sync_copy(acc_v, out_hbm.at[wid])


@functools.partial(jax.jit, static_argnames=())
def _center_loss(features, labels, centers):
    labels = labels.astype(jnp.int32)
    featw = features.reshape(BATCH // 2, WIDE)
    mask = jnp.broadcast_to(
        (labels >= HALF).astype(jnp.float32)[:, None],
        (BATCH, FEAT)).reshape(BATCH // 2, WIDE)
    lab3 = labels.reshape(NUM_WORKERS, NCHUNK, CHUNK)

    widen = pl.pallas_call(
        _widen_body,
        grid=(NW,),
        in_specs=[
            pl.BlockSpec((WCH, FEAT), lambda i: (i, 0)),
            pl.BlockSpec((WCH, FEAT), lambda i: (i + NW, 0)),
        ],
        out_specs=pl.BlockSpec((WCH, WIDE), lambda i: (i, 0)),
        out_shape=jax.ShapeDtypeStruct((HALF, WIDE), jnp.float32),
    )
    wide = widen(centers, centers)

    kern = pl.kernel(
        _loss_body,
        out_type=jax.ShapeDtypeStruct((NUM_WORKERS, LANES), jnp.float32),
        mesh=plsc.VectorSubcoreMesh(core_axis_name="c", subcore_axis_name="s"),
        scratch_types=[
            pltpu.VMEM((NCHUNK, CHUNK), jnp.int32),
            pltpu.VMEM((NCHUNK, CHUNK), jnp.int32),
            pltpu.VMEM((PPW, WIDE), jnp.float32),
            pltpu.VMEM((2, CHUNK, WIDE), jnp.float32),
            pltpu.VMEM((PPW, WIDE), jnp.float32),
            pltpu.VMEM((LANES,), jnp.float32),
            pltpu.SemaphoreType.DMA((2,)),
            pltpu.SemaphoreType.DMA,
            pltpu.SemaphoreType.DMA,
        ],
    )
    partials = kern(featw, lab3, mask, wide)
    return jnp.sum(partials) / BATCH


def kernel(features, labels, centers):
    return _center_loss(features, labels, centers)


# speedup vs baseline: 2.2294x; 1.6973x over previous
import functools

import jax
import jax.numpy as jnp
from jax import lax
from jax.experimental import pallas as pl
from jax.experimental.pallas import tpu as pltpu
from jax.experimental.pallas import tpu_sc as plsc

BATCH = 16384
FEAT = 64
LANES = 16
NUM_CORES = 2
NUM_SUBCORES = 16
NUM_WORKERS = NUM_CORES * NUM_SUBCORES
BPW = BATCH // NUM_WORKERS
CHB = 64
NCH = BPW // CHB
VECS_PER_ROW = FEAT // LANES


def _body(feat_hbm, lab_hbm, cent_hbm, out_hbm, idx_v, blk_v, feat_v,
          acc_v, gsems, fsem):
    wid = lax.axis_index("s") * NUM_CORES + lax.axis_index("c")
    base = wid * BPW
    pltpu.sync_copy(lab_hbm.at[pl.ds(base, BPW)], idx_v)
    fcopy = pltpu.async_copy(feat_hbm.at[pl.ds(base, BPW), :], feat_v, fsem)

    def fire(c, buf):
        def issue(v, _):
            labv = idx_v[pl.ds(c * CHB + v * LANES, LANES)]
            for k in range(LANES):
                pltpu.async_copy(cent_hbm.at[labv[k]],
                                 blk_v.at[buf, v * LANES + k],
                                 gsems.at[buf])
            return 0
        lax.fori_loop(0, CHB // LANES, issue, 0)

    def drain(buf):
        pltpu.make_async_copy(cent_hbm.at[pl.ds(0, CHB), :],
                              blk_v.at[buf], gsems.at[buf]).wait()

    def compute(c, buf, accs):
        def row(i, a):
            g = c * CHB + i
            out = []
            for l in range(VECS_PER_ROW):
                d = (feat_v[g, pl.ds(l * LANES, LANES)]
                     - blk_v[buf, i, pl.ds(l * LANES, LANES)])
                out.append(a[l] + d * d)
            return tuple(out)
        return lax.fori_loop(0, CHB, row, accs)

    zero = jnp.zeros((LANES,), jnp.float32)
    accs = (zero,) * VECS_PER_ROW
    fire(0, 0)
    fire(1, 1)
    fcopy.wait()
    for c in range(NCH):
        drain(c % 2)
        accs = compute(c, c % 2, accs)
        if c + 2 < NCH:
            fire(c + 2, c % 2)

    total = accs[0] + accs[1] + accs[2] + accs[3]
    acc_v[...] = total
    pltpu.sync_copy(acc_v, out_hbm.at[wid])


@functools.partial(jax.jit, static_argnames=())
def _center_loss(features, labels, centers):
    labels = labels.astype(jnp.int32)
    kern = pl.kernel(
        _body,
        out_type=jax.ShapeDtypeStruct((NUM_WORKERS, LANES), jnp.float32),
        mesh=plsc.VectorSubcoreMesh(core_axis_name="c", subcore_axis_name="s"),
        scratch_types=[
            pltpu.VMEM((BPW,), jnp.int32),
            pltpu.VMEM((2, CHB, FEAT), jnp.float32),
            pltpu.VMEM((BPW, FEAT), jnp.float32),
            pltpu.VMEM((LANES,), jnp.float32),
            pltpu.SemaphoreType.DMA((2,)),
            pltpu.SemaphoreType.DMA,
        ],
    )
    partials = kern(features, labels, centers)
    return jnp.sum(partials) / BATCH


def kernel(features, labels, centers):
    return _center_loss(features, labels, centers)


# speedup vs baseline: 2.2316x; 1.0010x over previous
import functools

import jax
import jax.numpy as jnp
from jax import lax
from jax.experimental import pallas as pl
from jax.experimental.pallas import tpu as pltpu
from jax.experimental.pallas import tpu_sc as plsc

BATCH = 16384
FEAT = 64
LANES = 16
NUM_CORES = 2
NUM_SUBCORES = 16
NUM_WORKERS = NUM_CORES * NUM_SUBCORES
BPW = BATCH // NUM_WORKERS
CHB = 128
NCH = BPW // CHB
VECS_PER_ROW = FEAT // LANES


def _body(feat_hbm, lab_hbm, cent_hbm, out_hbm, idx_v, blk_v, feat_v,
          acc_v, gsems, fsem):
    wid = lax.axis_index("s") * NUM_CORES + lax.axis_index("c")
    base = wid * BPW
    pltpu.sync_copy(lab_hbm.at[pl.ds(base, BPW)], idx_v)
    fcopy = pltpu.async_copy(feat_hbm.at[pl.ds(base, BPW), :], feat_v, fsem)

    def fire(c, buf):
        def issue(v, _):
            labv = idx_v[pl.ds(c * CHB + v * LANES, LANES)]
            for k in range(LANES):
                pltpu.async_copy(cent_hbm.at[labv[k]],
                                 blk_v.at[buf, v * LANES + k],
                                 gsems.at[buf])
            return 0
        lax.fori_loop(0, CHB // LANES, issue, 0)

    def drain(buf):
        pltpu.make_async_copy(cent_hbm.at[pl.ds(0, CHB), :],
                              blk_v.at[buf], gsems.at[buf]).wait()

    def compute(c, buf, accs):
        def row(i, a):
            g = c * CHB + i
            out = []
            for l in range(VECS_PER_ROW):
                d = (feat_v[g, pl.ds(l * LANES, LANES)]
                     - blk_v[buf, i, pl.ds(l * LANES, LANES)])
                out.append(a[l] + d * d)
            return tuple(out)
        return lax.fori_loop(0, CHB, row, accs)

    zero = jnp.zeros((LANES,), jnp.float32)
    accs = (zero,) * VECS_PER_ROW
    fire(0, 0)
    fire(1, 1)
    fcopy.wait()
    for c in range(NCH):
        drain(c % 2)
        accs = compute(c, c % 2, accs)
        if c + 2 < NCH:
            fire(c + 2, c % 2)

    total = accs[0] + accs[1] + accs[2] + accs[3]
    acc_v[...] = total
    pltpu.sync_copy(acc_v, out_hbm.at[wid])


@functools.partial(jax.jit, static_argnames=())
def _center_loss(features, labels, centers):
    labels = labels.astype(jnp.int32)
    kern = pl.kernel(
        _body,
        out_type=jax.ShapeDtypeStruct((NUM_WORKERS, LANES), jnp.float32),
        mesh=plsc.VectorSubcoreMesh(core_axis_name="c", subcore_axis_name="s"),
        scratch_types=[
            pltpu.VMEM((BPW,), jnp.int32),
            pltpu.VMEM((2, CHB, FEAT), jnp.float32),
            pltpu.VMEM((BPW, FEAT), jnp.float32),
            pltpu.VMEM((LANES,), jnp.float32),
            pltpu.SemaphoreType.DMA((2,)),
            pltpu.SemaphoreType.DMA,
        ],
    )
    partials = kern(features, labels, centers)
    return jnp.sum(partials) / BATCH


def kernel(features, labels, centers):
    return _center_loss(features, labels, centers)
